# baseline (speedup 1.0000x reference)
import numpy as np
import jax
import jax.numpy as jnp
from jax import lax
from jax.experimental import pallas as pl
from jax.experimental.pallas import tpu as pltpu
from jax.experimental.pallas import tpu_sc as plsc

N = 10000
E = 320000
DIM = 128
NSB = 50
NF = 128
CUTOFF = 5.0
LOG2 = float(np.log(2.0))

NUM_CORES = 2
NUM_SUBCORES = 16
E_PER_CORE = E // NUM_CORES
E_PER_WORKER = E_PER_CORE // NUM_SUBCORES
CH = 40
NCHUNK = E_PER_WORKER // CH
ZB = 200
NZB = N // ZB
ZB_PER_SUB = -(-NZB // NUM_SUBCORES)
WB = 624


def _ssp(v):
    return jnp.maximum(v, 0.0) + jnp.log(0.5 + 0.5 * jnp.exp(-jnp.abs(v)))


EBLK = 3200
RSUB = EBLK // 128


def _filter_compute(fblk, r_ref, w1_ref, b1_ref, w2_ref, b2_ref, o_ref):
    t = jnp.dot(fblk, w1_ref[...], preferred_element_type=jnp.float32)
    t = _ssp(t + b1_ref[...])
    t = jnp.dot(t, w2_ref[...], preferred_element_type=jnp.float32) + b2_ref[...]
    r = r_ref[0]
    c = 0.5 * (jnp.cos(r * (np.pi / CUTOFF)) + 1.0)
    c = jnp.where(r < CUTOFF, c, 0.0)
    c3 = lax.broadcast_in_dim(c, (RSUB, 128, NF), (0, 1))
    o_ref[...] = (t.reshape(RSUB, 128, NF) * c3).reshape(EBLK, NF).astype(jnp.bfloat16)


def _filter_body(f_ref, r_ref, w1_ref, b1_ref, w2_ref, b2_ref, o_ref):
    _filter_compute(f_ref[...], r_ref, w1_ref, b1_ref, w2_ref, b2_ref, o_ref)


_PERM = np.arange(NF).reshape(NF // 32, 2, 16).transpose(0, 2, 1).reshape(NF)


def _filter_net(f_ij, r_ij, w1, b1, w2, b2):
    w2 = w2[:, _PERM]
    b2 = b2[_PERM]
    return pl.pallas_call(
        _filter_body,
        grid=(E // EBLK,),
        in_specs=[
            pl.BlockSpec((EBLK, NSB), lambda i: (i, 0)),
            pl.BlockSpec((1, RSUB, 128), lambda i: (i, 0, 0)),
            pl.BlockSpec((NSB, NF), lambda i: (0, 0)),
            pl.BlockSpec((1, NF), lambda i: (0, 0)),
            pl.BlockSpec((NF, NF), lambda i: (0, 0)),
            pl.BlockSpec((1, NF), lambda i: (0, 0)),
        ],
        out_specs=pl.BlockSpec((EBLK, NF), lambda i: (i, 0)),
        out_shape=jax.ShapeDtypeStruct((E, NF), jnp.bfloat16),
    )(f_ij, r_ij.reshape(E // EBLK, RSUB, 128),
      w1, b1.reshape(1, NF), w2, b2.reshape(1, NF))


def _h_body(x_ref, w_ref, o_ref):
    o_ref[...] = jnp.dot(x_ref[...], w_ref[...], preferred_element_type=jnp.float32)


def _in2f(x, w):
    return pl.pallas_call(
        _h_body,
        out_shape=jax.ShapeDtypeStruct((N, NF), jnp.float32),
    )(x, w)


def _sc_body(h_hbm, w_hbm, indi_hbm, indj_hbm, zeros_hbm, out_hbm,
             acc, idxi_sp, idxj_sp, hbuf0, hbuf1, wbuf0, wbuf1,
             gsem0, gsem1, wsem0, wsem1, ssem0, ssem1, zsem):
    c = lax.axis_index("c")
    s = lax.axis_index("s")
    bufs = ((hbuf0, wbuf0, gsem0, wsem0, ssem0),
            (hbuf1, wbuf1, gsem1, wsem1, ssem1))

    wbase = c * E_PER_CORE + s * E_PER_WORKER
    wid = c * NUM_SUBCORES + s

    pltpu.async_copy(indi_hbm.at[wid], idxi_sp, zsem)
    pltpu.async_copy(indj_hbm.at[wid], idxj_sp, zsem)
    for t in range(ZB_PER_SUB):
        b = s + NUM_SUBCORES * t

        @pl.when(b < NZB)
        def _():
            r0 = pl.multiple_of(b * ZB, 8)
            pltpu.async_copy(zeros_hbm, acc.at[pl.ds(r0, ZB)], zsem)

    pltpu.make_async_copy(indi_hbm.at[wid], idxi_sp, zsem).wait()
    pltpu.make_async_copy(indj_hbm.at[wid], idxj_sp, zsem).wait()
    for t in range(ZB_PER_SUB):
        b = s + NUM_SUBCORES * t

        @pl.when(b < NZB)
        def _():
            r0 = pl.multiple_of(b * ZB, 8)
            pltpu.make_async_copy(zeros_hbm, acc.at[pl.ds(r0, ZB)], zsem).wait()

    plsc.subcore_barrier()

    def issue_loads(k, p):
        hb, wb, gs, ws, _ = bufs[p]
        pltpu.async_copy(h_hbm.at[idxj_sp.at[pl.ds(k * CH, CH)]], hb, gs)
        pltpu.async_copy(w_hbm.at[pl.ds(pl.multiple_of(wbase + k * CH, 8), CH)], wb, ws)

    def step(k, p):
        hb, wb, gs, ws, ss = bufs[p]
        hb_o, wb_o, _, _, ss_o = bufs[1 - p]
        pltpu.make_async_copy(h_hbm.at[idxj_sp.at[pl.ds(k * CH, CH)]], hb, gs).wait()
        pltpu.make_async_copy(
            w_hbm.at[pl.ds(pl.multiple_of(wbase + k * CH, 8), CH)], wb, ws).wait()

        @pl.when(k + 1 < NCHUNK)
        def _():
            @pl.when(k >= 1)
            def _():
                pltpu.make_async_copy(hb_o, acc.at[idxi_sp.at[pl.ds((k - 1) * CH, CH)]], ss_o).wait()

            issue_loads(k + 1, 1 - p)

        def mrow(r2, carry2):
            for dr in range(2):
                r = r2 * 2 + dr
                for l32 in range(NF // 32):
                    wi = wb[r, pl.ds(l32 * 16, 16)]
                    lo = jax.lax.bitcast_convert_type(wi * jnp.int32(65536), jnp.float32)
                    hi = jax.lax.bitcast_convert_type(
                        lax.bitwise_and(wi, jnp.int32(-65536)), jnp.float32)
                    sl0 = pl.ds(l32 * 32, 16)
                    sl1 = pl.ds(l32 * 32 + 16, 16)
                    hb[r, sl0] = hb[r, sl0] * lo
                    hb[r, sl1] = hb[r, sl1] * hi
            return carry2

        lax.fori_loop(0, CH // 2, mrow, 0)
        pltpu.async_copy(hb, acc.at[idxi_sp.at[pl.ds(k * CH, CH)]], ss, add=True)

    issue_loads(0, 0)

    def pair(g, carry):
        step(2 * g, 0)
        step(2 * g + 1, 1)
        return carry

    lax.fori_loop(0, NCHUNK // 2, pair, 0)
    if NCHUNK % 2:
        step(NCHUNK - 1, (NCHUNK - 1) % 2)
    pltpu.make_async_copy(hbuf0, acc.at[idxi_sp.at[pl.ds((NCHUNK - 1) * CH, CH)]], bufs[(NCHUNK - 1) % 2][4]).wait()
    pltpu.make_async_copy(hbuf1, acc.at[idxi_sp.at[pl.ds((NCHUNK - 2) * CH, CH)]], bufs[NCHUNK % 2][4]).wait()
    plsc.subcore_barrier()

    r0 = pl.multiple_of(WB * s, 8)
    pltpu.async_copy(acc.at[pl.ds(r0, WB)], out_hbm.at[c, pl.ds(r0, WB)], zsem)

    @pl.when(s == NUM_SUBCORES - 1)
    def _():
        t0 = WB * NUM_SUBCORES
        pltpu.async_copy(acc.at[pl.ds(t0, N - t0)], out_hbm.at[c, pl.ds(t0, N - t0)], zsem)

    pltpu.make_async_copy(acc.at[pl.ds(r0, WB)], out_hbm.at[c, pl.ds(r0, WB)], zsem).wait()

    @pl.when(s == NUM_SUBCORES - 1)
    def _():
        t0 = WB * NUM_SUBCORES
        pltpu.make_async_copy(acc.at[pl.ds(t0, N - t0)], out_hbm.at[c, pl.ds(t0, N - t0)], zsem).wait()


def _sc_aggregate(h, w_all, ind_i, ind_j):
    mesh = plsc.VectorSubcoreMesh(core_axis_name="c", subcore_axis_name="s")
    agg = pl.kernel(
        _sc_body,
        out_type=jax.ShapeDtypeStruct((NUM_CORES, N, NF), jnp.float32),
        mesh=mesh,
        scratch_types=[
            pltpu.VMEM_SHARED((N, NF), jnp.float32),
            pltpu.VMEM((E_PER_WORKER,), jnp.int32),
            pltpu.VMEM((E_PER_WORKER,), jnp.int32),
            pltpu.VMEM((CH, NF), jnp.float32),
            pltpu.VMEM((CH, NF), jnp.float32),
            pltpu.VMEM((CH, NF // 2), jnp.int32),
            pltpu.VMEM((CH, NF // 2), jnp.int32),
            pltpu.SemaphoreType.DMA,
            pltpu.SemaphoreType.DMA,
            pltpu.SemaphoreType.DMA,
            pltpu.SemaphoreType.DMA,
            pltpu.SemaphoreType.DMA,
            pltpu.SemaphoreType.DMA,
            pltpu.SemaphoreType.DMA,
        ],
    )
    zeros = jnp.zeros((ZB, NF), jnp.float32)

    w_i32 = jax.lax.bitcast_convert_type(
        w_all.reshape(E, NF // 2, 2), jnp.int32)
    return agg(h, w_i32, ind_i.reshape(NUM_CORES * NUM_SUBCORES, E_PER_WORKER),
               ind_j.reshape(NUM_CORES * NUM_SUBCORES, E_PER_WORKER), zeros)


def _out_body(p_ref, fw_ref, fb_ref, lw_ref, lb_ref, o_ref):
    a = p_ref[0] + p_ref[1]
    t = _ssp(jnp.dot(a, fw_ref[...], preferred_element_type=jnp.float32) + fb_ref[...])
    o_ref[...] = jnp.dot(t, lw_ref[...], preferred_element_type=jnp.float32) + lb_ref[...]


def _out_head(partials, fw, fb, lw, lb):
    return pl.pallas_call(
        _out_body,
        out_shape=jax.ShapeDtypeStruct((N, DIM), jnp.float32),
    )(partials, fw, fb.reshape(1, DIM), lw, lb.reshape(1, DIM))


def kernel(x, r_ij, f_ij, ind_i, ind_j, filt_W1, filt_b1, filt_W2, filt_b2,
           in2f_W, f2out_W, f2out_b, lin_W, lin_b):
    w_all = _filter_net(f_ij, r_ij, filt_W1, filt_b1, filt_W2, filt_b2)
    h = _in2f(x, in2f_W)
    partials = _sc_aggregate(h, w_all, ind_i, ind_j)
    return _out_head(partials, f2out_W, f2out_b, lin_W, lin_b)


# speedup vs baseline: 3.1640x; 3.1640x over previous
import numpy as np
import jax
import jax.numpy as jnp
from jax import lax
from jax.experimental import pallas as pl
from jax.experimental.pallas import tpu as pltpu
from jax.experimental.pallas import tpu_sc as plsc

N = 10000
E = 320000
DIM = 128
NSB = 50
NF = 128
CUTOFF = 5.0
LOG2 = float(np.log(2.0))

EH = E // 2
NUM_CORES = 2
NUM_SUBCORES = 16
NUM_WORKERS = NUM_CORES * NUM_SUBCORES
ROWS_PER_WORKER = EH // NUM_WORKERS
CH = 40
NCHUNK = ROWS_PER_WORKER // CH
ZB = 200
NZB = N // ZB
ZB_PER_SUB = -(-NZB // NUM_SUBCORES)
WB = 624


def _ssp(v):
    return jnp.maximum(v, 0.0) + jnp.log(0.5 + 0.5 * jnp.exp(-jnp.abs(v)))


EBLK = 3200
RSUB = EBLK // 128


def _filter_block(f, r, w1, b1, w2, b2):
    t = jnp.dot(f, w1, preferred_element_type=jnp.float32)
    t = _ssp(t + b1)
    t = jnp.dot(t, w2, preferred_element_type=jnp.float32) + b2
    c = 0.5 * (jnp.cos(r * (np.pi / CUTOFF)) + 1.0)
    c = jnp.where(r < CUTOFF, c, 0.0)
    c3 = lax.broadcast_in_dim(c, (RSUB, 128, NF), (0, 1))
    return (t.reshape(RSUB, 128, NF) * c3).reshape(EBLK, NF)


def _bf16_bits(x):
    xi = jax.lax.bitcast_convert_type(x, jnp.int32)
    lsb = lax.bitwise_and(lax.shift_right_logical(xi, 16), jnp.int32(1))
    return xi + lsb + jnp.int32(0x7FFF)


def _filter_body(fa_ref, fb_ref, ra_ref, rb_ref, w1_ref, b1_ref, w2_ref,
                 b2_ref, o_ref):
    ta = _filter_block(fa_ref[...], ra_ref[0], w1_ref[...], b1_ref[...],
                       w2_ref[...], b2_ref[...])
    tb = _filter_block(fb_ref[...], rb_ref[0], w1_ref[...], b1_ref[...],
                       w2_ref[...], b2_ref[...])
    lo = lax.shift_right_logical(_bf16_bits(ta), 16)
    hi = lax.bitwise_and(_bf16_bits(tb), jnp.int32(-65536))
    o_ref[...] = lax.bitwise_or(lo, hi)


def _filter_net(f_ij, r_ij, w1, b1, w2, b2):
    return pl.pallas_call(
        _filter_body,
        grid=(EH // EBLK,),
        in_specs=[
            pl.BlockSpec((EBLK, NSB), lambda i: (i, 0)),
            pl.BlockSpec((EBLK, NSB), lambda i: (i + EH // EBLK, 0)),
            pl.BlockSpec((1, RSUB, 128), lambda i: (i, 0, 0)),
            pl.BlockSpec((1, RSUB, 128), lambda i: (i + EH // EBLK, 0, 0)),
            pl.BlockSpec((NSB, NF), lambda i: (0, 0)),
            pl.BlockSpec((1, NF), lambda i: (0, 0)),
            pl.BlockSpec((NF, NF), lambda i: (0, 0)),
            pl.BlockSpec((1, NF), lambda i: (0, 0)),
        ],
        out_specs=pl.BlockSpec((EBLK, NF), lambda i: (i, 0)),
        out_shape=jax.ShapeDtypeStruct((EH, NF), jnp.int32),
    )(f_ij, f_ij, r_ij.reshape(E // EBLK, RSUB, 128),
      r_ij.reshape(E // EBLK, RSUB, 128),
      w1, b1.reshape(1, NF), w2, b2.reshape(1, NF))


def _h_body(x_ref, w_ref, o_ref):
    o_ref[...] = jnp.dot(x_ref[...], w_ref[...], preferred_element_type=jnp.float32)


def _in2f(x, w):
    return pl.pallas_call(
        _h_body,
        out_shape=jax.ShapeDtypeStruct((N, NF), jnp.float32),
    )(x, w)


def _sc_body(h_hbm, w_hbm, indi_hbm, indj_hbm, zeros_hbm, out_hbm,
             acc, idxj_sp, hba0, hba1, hbb0, hbb1, wbuf0, wbuf1,
             ia0, ia1, ib0, ib1,
             ga0, ga1, gb0, gb1, ws0, ws1, sa0, sa1, sb0, sb1, is0, is1, zsem):
    c = lax.axis_index("c")
    s = lax.axis_index("s")
    bufs = ((hba0, hbb0, wbuf0, ia0, ib0, ga0, gb0, ws0, sa0, sb0, is0),
            (hba1, hbb1, wbuf1, ia1, ib1, ga1, gb1, ws1, sa1, sb1, is1))

    wid = c * NUM_SUBCORES + s
    rbase = wid * ROWS_PER_WORKER

    ja = pl.multiple_of(rbase, 8)
    jb = pl.multiple_of(EH + rbase, 8)
    pltpu.async_copy(indj_hbm.at[pl.ds(ja, ROWS_PER_WORKER)],
                     idxj_sp.at[pl.ds(0, ROWS_PER_WORKER)], zsem)
    pltpu.async_copy(indj_hbm.at[pl.ds(jb, ROWS_PER_WORKER)],
                     idxj_sp.at[pl.ds(ROWS_PER_WORKER, ROWS_PER_WORKER)], zsem)
    for t in range(ZB_PER_SUB):
        b = s + NUM_SUBCORES * t

        @pl.when(b < NZB)
        def _():
            r0 = pl.multiple_of(b * ZB, 8)
            pltpu.async_copy(zeros_hbm, acc.at[pl.ds(r0, ZB)], zsem)

    pltpu.make_async_copy(indj_hbm.at[pl.ds(ja, ROWS_PER_WORKER)],
                          idxj_sp.at[pl.ds(0, ROWS_PER_WORKER)], zsem).wait()
    pltpu.make_async_copy(indj_hbm.at[pl.ds(jb, ROWS_PER_WORKER)],
                          idxj_sp.at[pl.ds(ROWS_PER_WORKER, ROWS_PER_WORKER)],
                          zsem).wait()
    for t in range(ZB_PER_SUB):
        b = s + NUM_SUBCORES * t

        @pl.when(b < NZB)
        def _():
            r0 = pl.multiple_of(b * ZB, 8)
            pltpu.make_async_copy(zeros_hbm, acc.at[pl.ds(r0, ZB)], zsem).wait()

    plsc.subcore_barrier()

    def issue_loads(k, p):
        hba, hbb, wb, ia, ib, ga, gb, ws, _, _, isem = bufs[p]
        pltpu.async_copy(h_hbm.at[idxj_sp.at[pl.ds(k * CH, CH)]], hba, ga)
        pltpu.async_copy(
            h_hbm.at[idxj_sp.at[pl.ds(ROWS_PER_WORKER + k * CH, CH)]], hbb, gb)
        row = pl.multiple_of(rbase + k * CH, 8)
        pltpu.async_copy(w_hbm.at[pl.ds(row, CH)], wb, ws)
        r8 = pl.multiple_of(rbase + k * CH, 8)
        pltpu.async_copy(indi_hbm.at[pl.ds(r8, CH)], ia, isem)
        pltpu.async_copy(indi_hbm.at[pl.ds(EH + r8, CH)], ib, isem)

    def wait_loads(k, p):
        hba, hbb, wb, ia, ib, ga, gb, ws, _, _, isem = bufs[p]
        pltpu.make_async_copy(h_hbm.at[idxj_sp.at[pl.ds(k * CH, CH)]], hba, ga).wait()
        pltpu.make_async_copy(
            h_hbm.at[idxj_sp.at[pl.ds(ROWS_PER_WORKER + k * CH, CH)]], hbb, gb).wait()
        row = pl.multiple_of(rbase + k * CH, 8)
        pltpu.make_async_copy(w_hbm.at[pl.ds(row, CH)], wb, ws).wait()
        r8 = pl.multiple_of(rbase + k * CH, 8)
        pltpu.make_async_copy(indi_hbm.at[pl.ds(r8, CH)], ia, isem).wait()
        pltpu.make_async_copy(indi_hbm.at[pl.ds(EH + r8, CH)], ib, isem).wait()

    def wait_scatters(p):
        hba, hbb, _, ia, ib, _, _, _, sa, sb, _ = bufs[p]
        pltpu.make_async_copy(hba, acc.at[ia], sa).wait()
        pltpu.make_async_copy(hbb, acc.at[ib], sb).wait()

    def step(k, p):
        hba, hbb, wb, ia, ib, ga, gb, ws, sa, sb, isem = bufs[p]
        wait_loads(k, p)

        @pl.when(k + 1 < NCHUNK)
        def _():
            @pl.when(k >= 1)
            def _():
                wait_scatters(1 - p)

            issue_loads(k + 1, 1 - p)

        def mrow(r, carry2):
            for l16 in range(NF // 16):
                sl = pl.ds(l16 * 16, 16)
                wi = wb[r, sl]
                lo = jax.lax.bitcast_convert_type(wi * jnp.int32(65536), jnp.float32)
                hi = jax.lax.bitcast_convert_type(
                    lax.bitwise_and(wi, jnp.int32(-65536)), jnp.float32)
                hba[r, sl] = hba[r, sl] * lo
                hbb[r, sl] = hbb[r, sl] * hi
            return carry2

        lax.fori_loop(0, CH, mrow, 0)
        pltpu.async_copy(hba, acc.at[ia], sa, add=True)
        pltpu.async_copy(hbb, acc.at[ib], sb, add=True)

    issue_loads(0, 0)

    def pair(g, carry):
        step(2 * g, 0)
        step(2 * g + 1, 1)
        return carry

    lax.fori_loop(0, NCHUNK // 2, pair, 0)
    if NCHUNK % 2:
        step(NCHUNK - 1, (NCHUNK - 1) % 2)
    wait_scatters((NCHUNK - 1) % 2)
    wait_scatters(NCHUNK % 2)
    plsc.subcore_barrier()

    r0 = pl.multiple_of(WB * s, 8)
    pltpu.async_copy(acc.at[pl.ds(r0, WB)], out_hbm.at[c, pl.ds(r0, WB)], zsem)

    @pl.when(s == NUM_SUBCORES - 1)
    def _():
        t0 = WB * NUM_SUBCORES
        pltpu.async_copy(acc.at[pl.ds(t0, N - t0)], out_hbm.at[c, pl.ds(t0, N - t0)], zsem)

    pltpu.make_async_copy(acc.at[pl.ds(r0, WB)], out_hbm.at[c, pl.ds(r0, WB)], zsem).wait()

    @pl.when(s == NUM_SUBCORES - 1)
    def _():
        t0 = WB * NUM_SUBCORES
        pltpu.make_async_copy(acc.at[pl.ds(t0, N - t0)], out_hbm.at[c, pl.ds(t0, N - t0)], zsem).wait()


def _sc_aggregate(h, w_all, ind_i, ind_j):
    mesh = plsc.VectorSubcoreMesh(core_axis_name="c", subcore_axis_name="s")
    dma = pltpu.SemaphoreType.DMA
    agg = pl.kernel(
        _sc_body,
        out_type=jax.ShapeDtypeStruct((NUM_CORES, N, NF), jnp.float32),
        mesh=mesh,
        scratch_types=[
            pltpu.VMEM_SHARED((N, NF), jnp.float32),
            pltpu.VMEM((2 * ROWS_PER_WORKER,), jnp.int32),
            pltpu.VMEM((CH, NF), jnp.float32),
            pltpu.VMEM((CH, NF), jnp.float32),
            pltpu.VMEM((CH, NF), jnp.float32),
            pltpu.VMEM((CH, NF), jnp.float32),
            pltpu.VMEM((CH, NF), jnp.int32),
            pltpu.VMEM((CH, NF), jnp.int32),
            pltpu.VMEM((CH,), jnp.int32),
            pltpu.VMEM((CH,), jnp.int32),
            pltpu.VMEM((CH,), jnp.int32),
            pltpu.VMEM((CH,), jnp.int32),
        ] + [dma] * 13,
    )
    zeros = jnp.zeros((ZB, NF), jnp.float32)

    return agg(h, w_all, ind_i, ind_j, zeros)


def _out_body(p_ref, fw_ref, fb_ref, lw_ref, lb_ref, o_ref):
    a = p_ref[0] + p_ref[1]
    t = _ssp(jnp.dot(a, fw_ref[...], preferred_element_type=jnp.float32) + fb_ref[...])
    o_ref[...] = jnp.dot(t, lw_ref[...], preferred_element_type=jnp.float32) + lb_ref[...]


def _out_head(partials, fw, fb, lw, lb):
    return pl.pallas_call(
        _out_body,
        out_shape=jax.ShapeDtypeStruct((N, DIM), jnp.float32),
    )(partials, fw, fb.reshape(1, DIM), lw, lb.reshape(1, DIM))


def kernel(x, r_ij, f_ij, ind_i, ind_j, filt_W1, filt_b1, filt_W2, filt_b2,
           in2f_W, f2out_W, f2out_b, lin_W, lin_b):
    w_all = _filter_net(f_ij, r_ij, filt_W1, filt_b1, filt_W2, filt_b2)
    h = _in2f(x, in2f_W)
    partials = _sc_aggregate(h, w_all, ind_i, ind_j)
    return _out_head(partials, f2out_W, f2out_b, lin_W, lin_b)
